# Initial kernel scaffold; baseline (speedup 1.0000x reference)
import jax
import jax.numpy as jnp
from jax.experimental import pallas as pl


def kernel(edge_index, etypes, embed, bases1, comp1, loop_w1, bias1, bases2, comp2, loop_w2, bias2):
    raise NotImplementedError("write your pallas kernel here")


# speedup vs baseline: 6.1192x; 6.1192x over previous
import functools

import jax
import jax.numpy as jnp
from jax import lax
from jax.experimental import pallas as pl
from jax.experimental.pallas import tpu as pltpu
from jax.experimental.pallas import tpu_sc as plsc

_pallas_call = pl.pallas_call
_pl_kernel = pl.kernel

_C = 80
_NW = 32
_NSUB = 16


def _expand_body(comp_ref, bases_ref, h_ref, v_ref):
    c = comp_ref[0, 0]
    w = jnp.sum(c[:, None, None] * bases_ref[...], axis=0)
    v_ref[0] = jnp.dot(h_ref[...], w, preferred_element_type=jnp.float32)


def _expand(h, bases_ext, comp_ext):
    rp, bp = comp_ext.shape
    n, d = h.shape
    return _pallas_call(
        _expand_body,
        grid=(rp,),
        in_specs=[
            pl.BlockSpec((1, 1, bp), lambda r: (r, 0, 0)),
            pl.BlockSpec((bp, d, d), lambda r: (0, 0, 0)),
            pl.BlockSpec((n, d), lambda r: (0, 0)),
        ],
        out_specs=pl.BlockSpec((1, n, d), lambda r: (r, 0, 0)),
        out_shape=jax.ShapeDtypeStruct((rp, n, d), jnp.float32),
    )(comp_ext.reshape(rp, 1, bp), bases_ext, h)


def _combine_body(agg_ref, vloop_ref, bias_ref, out_ref, *, relu):
    x = agg_ref[0] + agg_ref[1] + vloop_ref[0] + bias_ref[...]
    out_ref[...] = jnp.maximum(x, 0.0) if relu else x


def _combine(agg, v_ext, bias, relu):
    rp, n, d = v_ext.shape
    return _pallas_call(
        functools.partial(_combine_body, relu=relu),
        grid=(1,),
        in_specs=[
            pl.BlockSpec((2, n, d), lambda i: (0, 0, 0)),
            pl.BlockSpec((1, n, d), lambda i: (rp - 1, 0, 0)),
            pl.BlockSpec((1, d), lambda i: (0, 0)),
        ],
        out_specs=pl.BlockSpec((n, d), lambda i: (0, 0)),
        out_shape=jax.ShapeDtypeStruct((n, d), jnp.float32),
    )(agg, v_ext, bias.reshape(1, d))


def _edge_pass(src, dst, ety, v_flat, zeros_nd, n):
    e = src.shape[0]
    npad, d = zeros_nd.shape
    ept = e // _NW
    nchunks = ept // _C
    rpt = npad // _NSUB
    mesh = plsc.VectorSubcoreMesh(core_axis_name="c", subcore_axis_name="s")

    @functools.partial(
        _pl_kernel,
        out_type=jax.ShapeDtypeStruct((2 * npad, d), jnp.float32),
        mesh=mesh,
        scratch_types=[
            pltpu.VMEM((_C,), jnp.int32),
            pltpu.VMEM((_C,), jnp.int32),
            pltpu.VMEM((_C,), jnp.int32),
            pltpu.VMEM((_C,), jnp.int32),
            pltpu.VMEM((_C, d), jnp.float32),
            pltpu.VMEM_SHARED((npad, d), jnp.float32),
            pltpu.SemaphoreType.DMA,
        ],
    )
    def body(src_hbm, dst_hbm, ety_hbm, v_hbm, z_hbm, out_hbm,
             src_v, dst_v, ety_v, gidx_v, rows_v, agg_s, sem):
        cid = lax.axis_index("c")
        sid = lax.axis_index("s")
        wid = sid * 2 + cid

        pltpu.sync_copy(z_hbm.at[pl.ds(sid * rpt, rpt)],
                        agg_s.at[pl.ds(sid * rpt, rpt)])
        plsc.subcore_barrier()

        ebase = wid * ept

        def chunk(i, carry):
            off = pl.multiple_of(ebase + i * _C, 8)
            pltpu.sync_copy(src_hbm.at[pl.ds(off, _C)], src_v)
            pltpu.sync_copy(ety_hbm.at[pl.ds(off, _C)], ety_v)
            pltpu.sync_copy(dst_hbm.at[pl.ds(off, _C)], dst_v)
            for j in range(_C // 16):
                sl = pl.ds(j * 16, 16)
                gidx_v[sl] = ety_v[sl] * n + src_v[sl]
            pltpu.async_copy(v_hbm.at[gidx_v], rows_v, sem).wait()
            pltpu.sync_copy(rows_v, agg_s.at[dst_v], add=True)
            return carry

        lax.fori_loop(0, nchunks, chunk, 0)

        plsc.subcore_barrier()
        pltpu.sync_copy(agg_s.at[pl.ds(sid * rpt, rpt)],
                        out_hbm.at[pl.ds(cid * npad + sid * rpt, rpt)])

    return body(src, dst, ety, v_flat, zeros_nd)


def _layer(src, dst, ety, h, bases, comp, loop_w, bias, zeros_nd, relu):
    b = bases.shape[0]
    r = comp.shape[0]
    n, d = h.shape
    bases_ext = jnp.concatenate([bases, loop_w[None]], axis=0)
    loop_row = jnp.zeros((1, b + 1), comp.dtype).at[0, b].set(1.0)
    comp_ext = jnp.concatenate(
        [jnp.pad(comp, ((0, 0), (0, 1))), loop_row], axis=0)
    npad = zeros_nd.shape[0]
    v_ext = _expand(h, bases_ext, comp_ext)
    agg = _edge_pass(src, dst, ety, v_ext.reshape((r + 1) * n, d),
                     zeros_nd, n)
    return _combine(agg.reshape(2, npad, d)[:, :n], v_ext, bias, relu)


def kernel(edge_index, etypes, embed, bases1, comp1, loop_w1, bias1,
           bases2, comp2, loop_w2, bias2):
    src = edge_index[0].astype(jnp.int32)
    dst = edge_index[1].astype(jnp.int32)
    ety = etypes.astype(jnp.int32)
    n, d = embed.shape
    npad = ((n + _NSUB * 8 - 1) // (_NSUB * 8)) * (_NSUB * 8)
    zeros_nd = jnp.zeros((npad, d), jnp.float32)
    h1 = _layer(src, dst, ety, embed, bases1, comp1, loop_w1, bias1,
                zeros_nd, True)
    return _layer(src, dst, ety, h1, bases2, comp2, loop_w2, bias2,
                  zeros_nd, False)


# speedup vs baseline: 10.5519x; 1.7244x over previous
import functools

import jax
import jax.numpy as jnp
from jax import lax
from jax.experimental import pallas as pl
from jax.experimental.pallas import tpu as pltpu
from jax.experimental.pallas import tpu_sc as plsc

_pallas_call = pl.pallas_call
_pl_kernel = pl.kernel

_C = 80
_NW = 32
_NSUB = 16


def _expand_body(comp_ref, bases_ref, h_ref, v_ref):
    c = comp_ref[0, 0]
    w = jnp.sum(c[:, None, None] * bases_ref[...], axis=0)
    v_ref[0] = jnp.dot(h_ref[...], w, preferred_element_type=jnp.float32)


def _expand(h, bases_ext, comp_ext):
    rp, bp = comp_ext.shape
    n, d = h.shape
    return _pallas_call(
        _expand_body,
        grid=(rp,),
        in_specs=[
            pl.BlockSpec((1, 1, bp), lambda r: (r, 0, 0)),
            pl.BlockSpec((bp, d, d), lambda r: (0, 0, 0)),
            pl.BlockSpec((n, d), lambda r: (0, 0)),
        ],
        out_specs=pl.BlockSpec((1, n, d), lambda r: (r, 0, 0)),
        out_shape=jax.ShapeDtypeStruct((rp, n, d), jnp.float32),
    )(comp_ext.reshape(rp, 1, bp), bases_ext, h)


def _combine_body(agg_ref, vloop_ref, bias_ref, out_ref, *, relu):
    x = agg_ref[0] + agg_ref[1] + vloop_ref[0] + bias_ref[...]
    out_ref[...] = jnp.maximum(x, 0.0) if relu else x


def _combine(agg, v_ext, bias, relu):
    rp, n, d = v_ext.shape
    return _pallas_call(
        functools.partial(_combine_body, relu=relu),
        grid=(1,),
        in_specs=[
            pl.BlockSpec((2, n, d), lambda i: (0, 0, 0)),
            pl.BlockSpec((1, n, d), lambda i: (rp - 1, 0, 0)),
            pl.BlockSpec((1, d), lambda i: (0, 0)),
        ],
        out_specs=pl.BlockSpec((n, d), lambda i: (0, 0)),
        out_shape=jax.ShapeDtypeStruct((n, d), jnp.float32),
    )(agg, v_ext, bias.reshape(1, d))


def _edge_pass(ed_flat, v_flat, zeros_nd, n):
    e3 = ed_flat.shape[0]
    e = e3 // 3
    npad, d = zeros_nd.shape
    ept = e // _NW
    nchunks = ept // _C
    nch_total = e // _C
    npairs = (nchunks - 1) // 2
    edc = 3 * _C
    rpt = npad // _NSUB
    mesh = plsc.VectorSubcoreMesh(core_axis_name="c", subcore_axis_name="s")

    @functools.partial(
        _pl_kernel,
        out_type=jax.ShapeDtypeStruct((2 * npad, d), jnp.float32),
        mesh=mesh,
        scratch_types=[
            pltpu.VMEM((edc,), jnp.int32),
            pltpu.VMEM((edc,), jnp.int32),
            pltpu.VMEM((_C,), jnp.int32),
            pltpu.VMEM((_C,), jnp.int32),
            pltpu.VMEM((_C,), jnp.int32),
            pltpu.VMEM((_C,), jnp.int32),
            pltpu.VMEM((_C, d), jnp.float32),
            pltpu.VMEM((_C, d), jnp.float32),
            pltpu.VMEM_SHARED((npad, d), jnp.float32),
            pltpu.SemaphoreType.DMA,
            pltpu.SemaphoreType.DMA,
            pltpu.SemaphoreType.DMA,
            pltpu.SemaphoreType.DMA,
            pltpu.SemaphoreType.DMA,
            pltpu.SemaphoreType.DMA,
        ],
    )
    def body(ed_hbm, v_hbm, z_hbm, out_hbm,
             edb0, edb1, dstb0, dstb1, gidxb0, gidxb1, rowsb0, rowsb1,
             agg_s, si0, si1, sg0, sg1, ss0, ss1):
        cid = lax.axis_index("c")
        sid = lax.axis_index("s")
        wid = sid * 2 + cid
        edb = (edb0, edb1)
        dstb = (dstb0, dstb1)
        gidxb = (gidxb0, gidxb1)
        rowsb = (rowsb0, rowsb1)
        sem_i = (si0, si1)
        sem_g = (sg0, sg1)
        sem_s = (ss0, ss1)

        pltpu.sync_copy(z_hbm.at[pl.ds(sid * rpt, rpt)],
                        agg_s.at[pl.ds(sid * rpt, rpt)])
        plsc.subcore_barrier()

        cbase = wid * nchunks

        def idx_off(i):
            g = jnp.minimum(cbase + i, nch_total - 1)
            return pl.multiple_of(g * edc, 8)

        def issue_idx(i, s):
            pltpu.async_copy(ed_hbm.at[pl.ds(idx_off(i), edc)],
                             edb[s], sem_i[s])

        def wait_idx(s):
            pltpu.make_async_copy(ed_hbm.at[pl.ds(0, edc)],
                                  edb[s], sem_i[s]).wait()

        def prep(s):
            for j in range(_C // 16):
                sl = pl.ds(j * 16, 16)
                gidxb[s][sl] = edb[s][pl.ds(_C + j * 16, 16)] * n + edb[s][sl]
                dstb[s][sl] = edb[s][pl.ds(2 * _C + j * 16, 16)]

        def issue_gather(s):
            pltpu.async_copy(v_hbm.at[gidxb[s]], rowsb[s], sem_g[s])

        def wait_gather(s):
            pltpu.make_async_copy(v_hbm.at[gidxb[s]],
                                  rowsb[s], sem_g[s]).wait()

        def issue_scatter(s):
            return pltpu.async_copy(rowsb[s], agg_s.at[dstb[s]],
                                    sem_s[s], add=True)

        issue_idx(0, 0)
        issue_idx(1, 1)
        wait_idx(0)
        prep(0)
        issue_gather(0)
        wait_idx(1)
        prep(1)
        issue_gather(1)

        def pair(k, carry):
            i = 2 * k
            wait_gather(0)
            sc0 = issue_scatter(0)
            issue_idx(i + 2, 0)
            wait_gather(1)
            sc1 = issue_scatter(1)
            issue_idx(i + 3, 1)
            wait_idx(0)
            sc0.wait()
            prep(0)
            issue_gather(0)
            wait_idx(1)
            sc1.wait()
            prep(1)
            issue_gather(1)
            return carry

        lax.fori_loop(0, npairs, pair, 0)

        wait_gather(0)
        issue_scatter(0).wait()
        wait_gather(1)

        plsc.subcore_barrier()
        pltpu.sync_copy(agg_s.at[pl.ds(sid * rpt, rpt)],
                        out_hbm.at[pl.ds(cid * npad + sid * rpt, rpt)])

    return body(ed_flat, v_flat, zeros_nd)


def _layer(ed_flat, h, bases, comp, loop_w, bias, zeros_nd, relu):
    b = bases.shape[0]
    r = comp.shape[0]
    n, d = h.shape
    bases_ext = jnp.concatenate([bases, loop_w[None]], axis=0)
    loop_row = jnp.zeros((1, b + 1), comp.dtype).at[0, b].set(1.0)
    comp_ext = jnp.concatenate(
        [jnp.pad(comp, ((0, 0), (0, 1))), loop_row], axis=0)
    npad = zeros_nd.shape[0]
    v_ext = _expand(h, bases_ext, comp_ext)
    agg = _edge_pass(ed_flat, v_ext.reshape((r + 1) * n, d), zeros_nd, n)
    return _combine(agg.reshape(2, npad, d)[:, :n], v_ext, bias, relu)


def kernel(edge_index, etypes, embed, bases1, comp1, loop_w1, bias1,
           bases2, comp2, loop_w2, bias2):
    src = edge_index[0].astype(jnp.int32)
    dst = edge_index[1].astype(jnp.int32)
    ety = etypes.astype(jnp.int32)
    n, d = embed.shape
    npad = ((n + _NSUB * 8 - 1) // (_NSUB * 8)) * (_NSUB * 8)
    zeros_nd = jnp.zeros((npad, d), jnp.float32)
    ed_flat = jnp.stack(
        [src.reshape(-1, _C), ety.reshape(-1, _C), dst.reshape(-1, _C)],
        axis=1).reshape(-1)
    h1 = _layer(ed_flat, embed, bases1, comp1, loop_w1, bias1,
                zeros_nd, True)
    return _layer(ed_flat, h1, bases2, comp2, loop_w2, bias2,
                  zeros_nd, False)


# speedup vs baseline: 12.4384x; 1.1788x over previous
import functools

import jax
import jax.numpy as jnp
from jax import lax
from jax.experimental import pallas as pl
from jax.experimental.pallas import tpu as pltpu
from jax.experimental.pallas import tpu_sc as plsc

_pallas_call = pl.pallas_call
_pl_kernel = pl.kernel

_C = 80
_NW = 32
_NSUB = 16


def _expand_body(comp_ref, bases_ref, h_ref, v_ref):
    c = comp_ref[0, 0]
    w = jnp.sum(c[:, None, None] * bases_ref[...], axis=0)
    v_ref[0] = jnp.dot(h_ref[...], w, preferred_element_type=jnp.float32)


def _expand(h, bases_ext, comp_ext):
    rp, bp = comp_ext.shape
    n, d = h.shape
    return _pallas_call(
        _expand_body,
        grid=(rp,),
        in_specs=[
            pl.BlockSpec((1, 1, bp), lambda r: (r, 0, 0)),
            pl.BlockSpec((bp, d, d), lambda r: (0, 0, 0)),
            pl.BlockSpec((n, d), lambda r: (0, 0)),
        ],
        out_specs=pl.BlockSpec((1, n, d), lambda r: (r, 0, 0)),
        out_shape=jax.ShapeDtypeStruct((rp, n, d), jnp.float32),
    )(comp_ext.reshape(rp, 1, bp), bases_ext, h)


def _combine_body(agg_ref, vloop_ref, bias_ref, out_ref, *, relu):
    x = agg_ref[0] + agg_ref[1] + vloop_ref[0] + bias_ref[...]
    out_ref[...] = jnp.maximum(x, 0.0) if relu else x


def _combine(agg, v_ext, bias, relu):
    rp, n, d = v_ext.shape
    return _pallas_call(
        functools.partial(_combine_body, relu=relu),
        grid=(1,),
        in_specs=[
            pl.BlockSpec((2, n, d), lambda i: (0, 0, 0)),
            pl.BlockSpec((1, n, d), lambda i: (rp - 1, 0, 0)),
            pl.BlockSpec((1, d), lambda i: (0, 0)),
        ],
        out_specs=pl.BlockSpec((n, d), lambda i: (0, 0)),
        out_shape=jax.ShapeDtypeStruct((n, d), jnp.float32),
    )(agg, v_ext, bias.reshape(1, d))


def _edge_pass(ed_flat, v_flat, zeros_nd, n):
    e3 = ed_flat.shape[0]
    e = e3 // 3
    npad, d = zeros_nd.shape
    ept = e // _NW
    nchunks = ept // _C
    nch_total = e // _C
    ns = 4
    ngrp = nchunks // ns
    edc = 3 * _C
    rpt = npad // _NSUB
    mesh = plsc.VectorSubcoreMesh(core_axis_name="c", subcore_axis_name="s")

    @functools.partial(
        _pl_kernel,
        out_type=jax.ShapeDtypeStruct((2 * npad, d), jnp.float32),
        mesh=mesh,
        scratch_types=(
            [pltpu.VMEM((edc,), jnp.int32) for _ in range(ns)] +
            [pltpu.VMEM((_C,), jnp.int32) for _ in range(ns)] +
            [pltpu.VMEM((_C,), jnp.int32) for _ in range(ns)] +
            [pltpu.VMEM((_C, d), jnp.float32) for _ in range(ns)] +
            [pltpu.VMEM_SHARED((npad, d), jnp.float32)] +
            [pltpu.SemaphoreType.DMA for _ in range(3 * ns)]
        ),
    )
    def body(ed_hbm, v_hbm, z_hbm, out_hbm, *scr):
        cid = lax.axis_index("c")
        sid = lax.axis_index("s")
        wid = sid * 2 + cid
        edb = scr[0:ns]
        dstb = scr[ns:2 * ns]
        gidxb = scr[2 * ns:3 * ns]
        rowsb = scr[3 * ns:4 * ns]
        agg_s = scr[4 * ns]
        sem_i = scr[4 * ns + 1:4 * ns + 1 + ns]
        sem_g = scr[4 * ns + 1 + ns:4 * ns + 1 + 2 * ns]
        sem_s = scr[4 * ns + 1 + 2 * ns:4 * ns + 1 + 3 * ns]

        pltpu.sync_copy(z_hbm.at[pl.ds(sid * rpt, rpt)],
                        agg_s.at[pl.ds(sid * rpt, rpt)])
        plsc.subcore_barrier()

        cbase = wid * nchunks

        def idx_off(i):
            g = jnp.minimum(cbase + i, nch_total - 1)
            return pl.multiple_of(g * edc, 8)

        def issue_idx(i, s):
            pltpu.async_copy(ed_hbm.at[pl.ds(idx_off(i), edc)],
                             edb[s], sem_i[s])

        def wait_idx(s):
            pltpu.make_async_copy(ed_hbm.at[pl.ds(0, edc)],
                                  edb[s], sem_i[s]).wait()

        def prep(s):
            for j in range(_C // 16):
                sl = pl.ds(j * 16, 16)
                gidxb[s][sl] = edb[s][pl.ds(_C + j * 16, 16)] * n + edb[s][sl]
                dstb[s][sl] = edb[s][pl.ds(2 * _C + j * 16, 16)]

        def issue_gather(s):
            pltpu.async_copy(v_hbm.at[gidxb[s]], rowsb[s], sem_g[s])

        def wait_gather(s):
            pltpu.make_async_copy(v_hbm.at[gidxb[s]],
                                  rowsb[s], sem_g[s]).wait()

        def issue_scatter(s):
            return pltpu.async_copy(rowsb[s], agg_s.at[dstb[s]],
                                    sem_s[s], add=True)

        for s in range(ns):
            issue_idx(s, s)
        for s in range(ns):
            wait_idx(s)
            prep(s)
            issue_gather(s)

        def grp(k, carry):
            i = ns * k
            scs = []
            for s in range(ns):
                wait_gather(s)
                scs.append(issue_scatter(s))
                issue_idx(i + ns + s, s)
            for s in range(ns):
                wait_idx(s)
                scs[s].wait()
                prep(s)
                issue_gather(s)
            return carry

        lax.fori_loop(0, ngrp, grp, 0)

        ntail = nchunks - ns * ngrp
        last = []
        for s in range(ns):
            wait_gather(s)
            if s < ntail:
                last.append(issue_scatter(s))
        for h in last:
            h.wait()

        plsc.subcore_barrier()
        pltpu.sync_copy(agg_s.at[pl.ds(sid * rpt, rpt)],
                        out_hbm.at[pl.ds(cid * npad + sid * rpt, rpt)])

    return body(ed_flat, v_flat, zeros_nd)


def _layer(ed_flat, h, bases, comp, loop_w, bias, zeros_nd, relu):
    b = bases.shape[0]
    r = comp.shape[0]
    n, d = h.shape
    bases_ext = jnp.concatenate([bases, loop_w[None]], axis=0)
    loop_row = jnp.zeros((1, b + 1), comp.dtype).at[0, b].set(1.0)
    comp_ext = jnp.concatenate(
        [jnp.pad(comp, ((0, 0), (0, 1))), loop_row], axis=0)
    npad = zeros_nd.shape[0]
    v_ext = _expand(h, bases_ext, comp_ext)
    agg = _edge_pass(ed_flat, v_ext.reshape((r + 1) * n, d), zeros_nd, n)
    return _combine(agg.reshape(2, npad, d)[:, :n], v_ext, bias, relu)


def kernel(edge_index, etypes, embed, bases1, comp1, loop_w1, bias1,
           bases2, comp2, loop_w2, bias2):
    src = edge_index[0].astype(jnp.int32)
    dst = edge_index[1].astype(jnp.int32)
    ety = etypes.astype(jnp.int32)
    n, d = embed.shape
    npad = ((n + _NSUB * 8 - 1) // (_NSUB * 8)) * (_NSUB * 8)
    zeros_nd = jnp.zeros((npad, d), jnp.float32)
    ed_flat = jnp.stack(
        [src.reshape(-1, _C), ety.reshape(-1, _C), dst.reshape(-1, _C)],
        axis=1).reshape(-1)
    h1 = _layer(ed_flat, embed, bases1, comp1, loop_w1, bias1,
                zeros_nd, True)
    return _layer(ed_flat, h1, bases2, comp2, loop_w2, bias2,
                  zeros_nd, False)
